# baseline (speedup 1.0000x reference)
import functools

import jax
import jax.numpy as jnp
from jax import lax
from jax.experimental import pallas as pl
from jax.experimental.pallas import tpu as pltpu
from jax.experimental.pallas import tpu_sc as plsc

PRE = 128
D = 18432
B = 2048
NW = 32
PER_W = B // NW
K = 8

_mesh = plsc.VectorSubcoreMesh(core_axis_name="c", subcore_axis_name="s")


@functools.partial(
    pl.kernel,
    mesh=_mesh,
    out_type=jax.ShapeDtypeStruct((B, D), jnp.float32),
    scratch_types=[
        pltpu.VMEM((PER_W,), jnp.int32),
        pltpu.SemaphoreType.DMA,
    ],
)
def _gather_kernel(idx_hbm, table_hbm, out_hbm, idx_v, sem):
    wid = lax.axis_index("s") * 2 + lax.axis_index("c")
    base = wid * PER_W
    pltpu.sync_copy(idx_hbm.at[pl.ds(base, PER_W)], idx_v)

    def body(g, carry):
        idx16 = idx_v[pl.ds(g * 16, 16)]
        for h in range(16 // K):
            for j in range(K):
                row = idx16[h * K + j]
                pltpu.async_copy(
                    table_hbm.at[row],
                    out_hbm.at[base + g * 16 + h * K + j],
                    sem,
                )
            for j in range(K):
                pltpu.make_async_copy(table_hbm.at[0], out_hbm.at[base], sem).wait()
        return carry

    lax.fori_loop(0, PER_W // 16, body, 0)


def kernel(prefix, embedding):
    idx = prefix.reshape(B)
    out = _gather_kernel(idx, embedding)
    return out.reshape(prefix.shape[0], prefix.shape[1], D)


# speedup vs baseline: 36.3785x; 36.3785x over previous
import functools

import jax
import jax.numpy as jnp
from jax import lax
from jax.experimental import pallas as pl
from jax.experimental.pallas import tpu as pltpu
from jax.experimental.pallas import tpu_sc as plsc

PRE = 128
D = 18432
B = 2048
NW = 32
PER_W = B // NW
C = 2
NCH = PER_W // C
NB = 3

_mesh = plsc.VectorSubcoreMesh(core_axis_name="c", subcore_axis_name="s")


@functools.partial(
    pl.kernel,
    mesh=_mesh,
    out_type=jax.ShapeDtypeStruct((B, D), jnp.float32),
    scratch_types=[
        pltpu.VMEM((NCH, C), jnp.int32),
        pltpu.VMEM((NB, C, D), jnp.float32),
        pltpu.SemaphoreType.DMA,
        pltpu.SemaphoreType.DMA,
        pltpu.SemaphoreType.DMA,
        pltpu.SemaphoreType.DMA,
        pltpu.SemaphoreType.DMA,
        pltpu.SemaphoreType.DMA,
    ],
)
def _gather_kernel(idx_hbm, table_hbm, out_hbm, idx_v, bufs, g0, g1, g2, p0, p1, p2):
    wid = lax.axis_index("s") * 2 + lax.axis_index("c")
    base = wid * PER_W
    pltpu.sync_copy(idx_hbm.at[pl.ds(wid * NCH, NCH)], idx_v)

    gsems = (g0, g1, g2)
    psems = (p0, p1, p2)

    for b in range(NB):
        pltpu.async_copy(table_hbm.at[idx_v.at[b]], bufs.at[b], gsems[b])

    def body(i, carry):
        for b in range(NB):
            k = i * NB + b
            buf, gs, ps = bufs.at[b], gsems[b], psems[b]
            pltpu.make_async_copy(table_hbm.at[idx_v.at[0]], buf, gs).wait()
            pltpu.async_copy(buf, out_hbm.at[pl.ds(base + k * C, C)], ps)

            @pl.when(k + NB < NCH)
            def _():
                pltpu.make_async_copy(buf, out_hbm.at[pl.ds(base, C)], ps).wait()
                pltpu.async_copy(table_hbm.at[idx_v.at[k + NB]], buf, gs)

        return carry

    lax.fori_loop(0, NCH // NB, body, 0)
    for k in range(NB * (NCH // NB), NCH):
        b = k % NB
        buf, gs, ps = bufs.at[b], gsems[b], psems[b]
        pltpu.make_async_copy(table_hbm.at[idx_v.at[0]], buf, gs).wait()
        pltpu.async_copy(buf, out_hbm.at[pl.ds(base + k * C, C)], ps)

    for b in range(NB):
        pltpu.make_async_copy(bufs.at[b], out_hbm.at[pl.ds(base, C)], psems[b]).wait()


def kernel(prefix, embedding):
    idx = prefix.reshape(NW * NCH, C)
    out = _gather_kernel(idx, embedding)
    return out.reshape(prefix.shape[0], prefix.shape[1], D)


# speedup vs baseline: 54.8678x; 1.5082x over previous
import functools

import jax
import jax.numpy as jnp
from jax import lax
from jax.experimental import pallas as pl
from jax.experimental.pallas import tpu as pltpu
from jax.experimental.pallas import tpu_sc as plsc

PRE = 128
D = 18432
B = 2048
NW = 32
W_LO = 512
W_HI = 640
NG = B // 16

_mesh = plsc.VectorSubcoreMesh(core_axis_name="c", subcore_axis_name="s")


@functools.partial(
    pl.kernel,
    mesh=_mesh,
    out_type=jax.ShapeDtypeStruct((B, D), jnp.float32),
    scratch_types=[
        pltpu.VMEM((B,), jnp.int32),
        pltpu.VMEM((PRE, W_HI), jnp.float32),
        pltpu.SemaphoreType.DMA,
    ],
)
def _gather_kernel(idx_hbm, table_hbm, out_hbm, idx_v, table_v, sem):
    wid = lax.axis_index("s") * 2 + lax.axis_index("c")
    pltpu.sync_copy(idx_hbm, idx_v)

    def run(width, col0):
        col0 = pl.multiple_of(col0, 128)
        pltpu.sync_copy(
            table_hbm.at[:, pl.ds(col0, width)], table_v.at[:, pl.ds(0, width)]
        )

        def body(g, carry):
            idx16 = idx_v[pl.ds(g * 16, 16)]
            for j in range(16):
                row = idx16[j]
                pltpu.async_copy(
                    table_v.at[row, pl.ds(0, width)],
                    out_hbm.at[g * 16 + j, pl.ds(col0, width)],
                    sem,
                )
            for j in range(16):
                pltpu.make_async_copy(
                    table_v.at[0, pl.ds(0, width)],
                    out_hbm.at[0, pl.ds(col0, width)],
                    sem,
                ).wait()
            return carry

        lax.fori_loop(0, NG, body, 0)

    @pl.when(wid < 16)
    def _():
        run(W_LO, wid * W_LO)

    @pl.when(wid >= 16)
    def _():
        run(W_HI, 16 * W_LO + (wid - 16) * W_HI)


def kernel(prefix, embedding):
    idx = prefix.reshape(B)
    out = _gather_kernel(idx, embedding)
    return out.reshape(prefix.shape[0], prefix.shape[1], D)


# speedup vs baseline: 60.9556x; 1.1110x over previous
import functools

import jax
import jax.numpy as jnp
from jax import lax
from jax.experimental import pallas as pl
from jax.experimental.pallas import tpu as pltpu
from jax.experimental.pallas import tpu_sc as plsc

PRE = 128
D = 18432
B = 2048
NW = 32
W_LO = 512
W_HI = 640
NG = B // 16

_mesh = plsc.VectorSubcoreMesh(core_axis_name="c", subcore_axis_name="s")


@functools.partial(
    pl.kernel,
    mesh=_mesh,
    out_type=jax.ShapeDtypeStruct((B, D), jnp.float32),
    scratch_types=[
        pltpu.VMEM((B,), jnp.int32),
        pltpu.VMEM((PRE, W_HI), jnp.float32),
        pltpu.SemaphoreType.DMA,
    ],
)
def _gather_kernel(idx_hbm, table_hbm, out_hbm, idx_v, table_v, sem):
    wid = lax.axis_index("s") * 2 + lax.axis_index("c")
    pltpu.sync_copy(idx_hbm, idx_v)

    def run(width, col0):
        col0 = pl.multiple_of(col0, 128)
        pltpu.sync_copy(
            table_hbm.at[:, pl.ds(col0, width)], table_v.at[:, pl.ds(0, width)]
        )

        def body(g, carry):
            idx16 = idx_v[pl.ds(g * 16, 16)]
            for j in range(16):
                row = idx16[j]
                pltpu.async_copy(
                    table_v.at[row, pl.ds(0, width)],
                    out_hbm.at[g * 16 + j, pl.ds(col0, width)],
                    sem,
                )

            @pl.when(g > 0)
            def _():
                for j in range(16):
                    pltpu.make_async_copy(
                        table_v.at[0, pl.ds(0, width)],
                        out_hbm.at[0, pl.ds(col0, width)],
                        sem,
                    ).wait()

            return carry

        lax.fori_loop(0, NG, body, 0)
        for j in range(16):
            pltpu.make_async_copy(
                table_v.at[0, pl.ds(0, width)],
                out_hbm.at[0, pl.ds(col0, width)],
                sem,
            ).wait()

    @pl.when(wid < 16)
    def _():
        run(W_LO, wid * W_LO)

    @pl.when(wid >= 16)
    def _():
        run(W_HI, 16 * W_LO + (wid - 16) * W_HI)


def kernel(prefix, embedding):
    idx = prefix.reshape(B)
    out = _gather_kernel(idx, embedding)
    return out.reshape(prefix.shape[0], prefix.shape[1], D)
